# baseline (speedup 1.0000x reference)
import jax
import jax.numpy as jnp
from jax.experimental import pallas as pl
from jax.experimental.pallas import tpu as pltpu

_N = 2048
_D = 768
_H = 128
_H2 = 2 * _H
_GH = 64
_E = 16
_EH = _E * _H
_TN = 1024
_NT = _N // _TN
_BALANCE_COEF = 0.01
_NEG = -1e30


def _moe_body(x_ref, gw1_ref, gb1_ref, gw2_ref, gb2_ref,
              w1_any, b1_ref, w2_any, b2_ref, w3_any, b3_ref,
              out_ref, usage_ref, loss_ref,
              h2s_ref, comb_ref, w1f_ref, w2f_ref, w3f_ref,
              w1c_ref, b1c_ref, w2p_ref, w3c_ref,
              sem1, sem2, sem3):
    i = pl.program_id(0)

    @pl.when(i == 0)
    def _gating_all():
        pltpu.make_async_copy(w1_any, w1f_ref, sem1).start()
        pltpu.make_async_copy(w2_any, w2f_ref, sem2).start()
        pltpu.make_async_copy(w3_any, w3f_ref, sem3).start()

        x = x_ref[...]
        gh = jnp.maximum(
            jnp.dot(x, gw1_ref[...], preferred_element_type=jnp.float32)
            + gb1_ref[...], 0.0)
        logits = (jnp.dot(gh, gw2_ref[...],
                          preferred_element_type=jnp.float32)
                  + gb2_ref[...])
        lane = jax.lax.broadcasted_iota(jnp.int32, (_N, _E), 1)
        m0 = jnp.max(logits, axis=1, keepdims=True)
        idx0 = jnp.min(jnp.where(logits == m0, lane, _E),
                       axis=1, keepdims=True)
        mask0 = lane == idx0
        lm = jnp.where(mask0, _NEG, logits)
        m1 = jnp.max(lm, axis=1, keepdims=True)
        idx1 = jnp.min(jnp.where(lm == m1, lane, _E), axis=1, keepdims=True)
        mask1 = lane == idx1
        w1r = 1.0 / (1.0 + jnp.exp(m0 - m1))
        w0r = 1.0 - w1r
        comb_ref[...] = jnp.where(mask0, w0r, 0.0) + jnp.where(mask1, w1r, 0.0)

        sel = mask0.astype(jnp.float32) + mask1.astype(jnp.float32)
        usage = jnp.sum(sel, axis=0) / _N
        usage_ref[...] = usage.reshape(1, _E)
        loss_ref[...] = (jnp.mean((usage - 1.0 / _E) ** 2)
                         * _BALANCE_COEF).reshape(1, 1)

    @pl.when(i == 1)
    def _prep_weights():
        pltpu.make_async_copy(w1_any, w1f_ref, sem1).wait()
        pltpu.make_async_copy(w2_any, w2f_ref, sem2).wait()
        pltpu.make_async_copy(w3_any, w3f_ref, sem3).wait()
        for e in range(_E):
            w1c_ref[:, e * _H:(e + 1) * _H] = w1f_ref[e].astype(jnp.bfloat16)
            w3c_ref[e * _H:(e + 1) * _H, :] = w3f_ref[e]
            b1c_ref[0:1, e * _H:(e + 1) * _H] = b1_ref[e:e + 1, :]
        zero = jnp.zeros((_H, _H), jnp.bfloat16)
        for p in range(_E // 2):
            w2p_ref[p, 0:_H, 0:_H] = w2f_ref[2 * p].astype(jnp.bfloat16)
            w2p_ref[p, 0:_H, _H:_H2] = zero
            w2p_ref[p, _H:_H2, 0:_H] = zero
            w2p_ref[p, _H:_H2, _H:_H2] = w2f_ref[2 * p + 1].astype(
                jnp.bfloat16)

    @pl.when(i > 0)
    def _ffn():
        t = i - 1
        x = x_ref[pl.ds(t * _TN, _TN), :]
        combine = comb_ref[pl.ds(t * _TN, _TN), :]
        lane = jax.lax.broadcasted_iota(jnp.int32, (_TN, _E), 1)
        lane2 = jax.lax.broadcasted_iota(jnp.int32, (_TN, _H2), 1)
        h1b = jnp.maximum(
            jnp.dot(x.astype(jnp.bfloat16), w1c_ref[...],
                    preferred_element_type=jnp.float32)
            + b1c_ref[...], 0.0).astype(jnp.bfloat16)
        for p in range(_E // 2):
            c0 = jnp.sum(jnp.where(lane == 2 * p, combine, 0.0),
                         axis=1, keepdims=True)
            c1 = jnp.sum(jnp.where(lane == 2 * p + 1, combine, 0.0),
                         axis=1, keepdims=True)
            cpair = jnp.where(lane2 < _H, c0, c1)
            bpair = jnp.concatenate(
                [b2_ref[2 * p:2 * p + 1, :], b2_ref[2 * p + 1:2 * p + 2, :]],
                axis=1)
            h2 = jnp.maximum(
                jnp.dot(h1b[:, p * _H2:(p + 1) * _H2], w2p_ref[p],
                        preferred_element_type=jnp.float32)
                + bpair, 0.0)
            h2s_ref[:, p * _H2:(p + 1) * _H2] = cpair * h2
        out_ref[...] = (
            jnp.dot(h2s_ref[...], w3c_ref[...],
                    preferred_element_type=jnp.float32)
            + jnp.dot(combine, b3_ref[...],
                      preferred_element_type=jnp.float32))


def kernel(x, gate_W1, gate_b1, gate_W2, gate_b2, W1, b1, W2, b2, W3, b3):
    out, usage, loss = pl.pallas_call(
        _moe_body,
        grid=(_NT + 1,),
        in_specs=[
            pl.BlockSpec((_N, _D), lambda i: (0, 0)),
            pl.BlockSpec((_D, _GH), lambda i: (0, 0)),
            pl.BlockSpec((1, _GH), lambda i: (0, 0)),
            pl.BlockSpec((_GH, _E), lambda i: (0, 0)),
            pl.BlockSpec((1, _E), lambda i: (0, 0)),
            pl.BlockSpec(memory_space=pl.ANY),
            pl.BlockSpec((_E, _H), lambda i: (0, 0)),
            pl.BlockSpec(memory_space=pl.ANY),
            pl.BlockSpec((_E, _H), lambda i: (0, 0)),
            pl.BlockSpec(memory_space=pl.ANY),
            pl.BlockSpec((_E, _D), lambda i: (0, 0)),
        ],
        out_specs=[
            pl.BlockSpec((_TN, _D),
                         lambda i: (jnp.maximum(i - 1, 0), 0)),
            pl.BlockSpec((1, _E), lambda i: (0, 0)),
            pl.BlockSpec((1, 1), lambda i: (0, 0)),
        ],
        out_shape=[
            jax.ShapeDtypeStruct((_N, _D), jnp.float32),
            jax.ShapeDtypeStruct((1, _E), jnp.float32),
            jax.ShapeDtypeStruct((1, 1), jnp.float32),
        ],
        scratch_shapes=[
            pltpu.VMEM((_TN, _EH), jnp.float32),
            pltpu.VMEM((_N, _E), jnp.float32),
            pltpu.VMEM((_E, _D, _H), jnp.float32),
            pltpu.VMEM((_E, _H, _H), jnp.float32),
            pltpu.VMEM((_E, _H, _D), jnp.float32),
            pltpu.VMEM((_D, _EH), jnp.bfloat16),
            pltpu.VMEM((1, _EH), jnp.float32),
            pltpu.VMEM((_E // 2, _H2, _H2), jnp.bfloat16),
            pltpu.VMEM((_EH, _D), jnp.float32),
            pltpu.SemaphoreType.DMA,
            pltpu.SemaphoreType.DMA,
            pltpu.SemaphoreType.DMA,
        ],
    )(x, gate_W1, gate_b1.reshape(1, _GH), gate_W2, gate_b2.reshape(1, _E),
      W1, b1, W2, b2, W3, b3)
    return out, loss.reshape(()), usage.reshape(_E)


# speedup vs baseline: 1.0295x; 1.0295x over previous
import jax
import jax.numpy as jnp
from jax.experimental import pallas as pl
from jax.experimental.pallas import tpu as pltpu

_N = 2048
_D = 768
_H = 128
_H2 = 2 * _H
_GH = 64
_E = 16
_EH = _E * _H
_TN = 1024
_NT = _N // _TN
_BALANCE_COEF = 0.01
_NEG = -1e30


def _moe_body(x_ref, gw1_ref, gb1_ref, gw2_ref, gb2_ref,
              w1_any, b1_ref, w2_any, b2_ref, w3_any, b3_ref,
              out_ref, usage_ref, loss_ref,
              h2s_ref, comb_ref, w1f_ref, w2f_ref, w3f_ref,
              w1c_ref, b1c_ref, w2p_ref, w3c_ref,
              sem1, sem2, sem3):
    i = pl.program_id(0)

    @pl.when(i == 0)
    def _gating_all():
        pltpu.make_async_copy(w1_any, w1f_ref, sem1).start()
        pltpu.make_async_copy(w2_any, w2f_ref, sem2).start()
        pltpu.make_async_copy(w3_any, w3f_ref, sem3).start()

        x = x_ref[...]
        gh = jnp.maximum(
            jnp.dot(x, gw1_ref[...], preferred_element_type=jnp.float32)
            + gb1_ref[...], 0.0)
        logits = (jnp.dot(gh, gw2_ref[...],
                          preferred_element_type=jnp.float32)
                  + gb2_ref[...])
        lane = jax.lax.broadcasted_iota(jnp.int32, (_N, _E), 1)
        m0 = jnp.max(logits, axis=1, keepdims=True)
        idx0 = jnp.min(jnp.where(logits == m0, lane, _E),
                       axis=1, keepdims=True)
        mask0 = lane == idx0
        lm = jnp.where(mask0, _NEG, logits)
        m1 = jnp.max(lm, axis=1, keepdims=True)
        idx1 = jnp.min(jnp.where(lm == m1, lane, _E), axis=1, keepdims=True)
        mask1 = lane == idx1
        w1r = 1.0 / (1.0 + jnp.exp(m0 - m1))
        w0r = 1.0 - w1r
        comb_ref[...] = jnp.where(mask0, w0r, 0.0) + jnp.where(mask1, w1r, 0.0)

        sel = mask0.astype(jnp.float32) + mask1.astype(jnp.float32)
        usage = jnp.sum(sel, axis=0) / _N
        usage_ref[...] = usage.reshape(1, _E)
        loss_ref[...] = (jnp.mean((usage - 1.0 / _E) ** 2)
                         * _BALANCE_COEF).reshape(1, 1)

    @pl.when(i == 1)
    def _prep_weights():
        pltpu.make_async_copy(w1_any, w1f_ref, sem1).wait()
        pltpu.make_async_copy(w2_any, w2f_ref, sem2).wait()
        pltpu.make_async_copy(w3_any, w3f_ref, sem3).wait()
        for e in range(_E):
            w1c_ref[:, e * _H:(e + 1) * _H] = w1f_ref[e].astype(jnp.bfloat16)
            w3c_ref[e * _H:(e + 1) * _H, :] = w3f_ref[e].astype(jnp.bfloat16)
            b1c_ref[0:1, e * _H:(e + 1) * _H] = b1_ref[e:e + 1, :]
        zero = jnp.zeros((_H, _H), jnp.bfloat16)
        for p in range(_E // 2):
            w2p_ref[p, 0:_H, 0:_H] = w2f_ref[2 * p].astype(jnp.bfloat16)
            w2p_ref[p, 0:_H, _H:_H2] = zero
            w2p_ref[p, _H:_H2, 0:_H] = zero
            w2p_ref[p, _H:_H2, _H:_H2] = w2f_ref[2 * p + 1].astype(
                jnp.bfloat16)

    @pl.when(i > 0)
    def _ffn():
        t = i - 1
        x = x_ref[pl.ds(t * _TN, _TN), :]
        combine = comb_ref[pl.ds(t * _TN, _TN), :]
        lane = jax.lax.broadcasted_iota(jnp.int32, (_TN, _E), 1)
        lane2 = jax.lax.broadcasted_iota(jnp.int32, (_TN, _H2), 1)
        h1b = jnp.maximum(
            jnp.dot(x.astype(jnp.bfloat16), w1c_ref[...],
                    preferred_element_type=jnp.float32)
            + b1c_ref[...], 0.0).astype(jnp.bfloat16)
        for p in range(_E // 2):
            c0 = jnp.sum(jnp.where(lane == 2 * p, combine, 0.0),
                         axis=1, keepdims=True)
            c1 = jnp.sum(jnp.where(lane == 2 * p + 1, combine, 0.0),
                         axis=1, keepdims=True)
            cpair = jnp.where(lane2 < _H, c0, c1)
            bpair = jnp.concatenate(
                [b2_ref[2 * p:2 * p + 1, :], b2_ref[2 * p + 1:2 * p + 2, :]],
                axis=1)
            h2 = jnp.maximum(
                jnp.dot(h1b[:, p * _H2:(p + 1) * _H2], w2p_ref[p],
                        preferred_element_type=jnp.float32)
                + bpair, 0.0)
            h2s_ref[:, p * _H2:(p + 1) * _H2] = (
                cpair * h2).astype(jnp.bfloat16)
        out_ref[...] = (
            jnp.dot(h2s_ref[...], w3c_ref[...],
                    preferred_element_type=jnp.float32)
            + jnp.dot(combine, b3_ref[...],
                      preferred_element_type=jnp.float32))


def kernel(x, gate_W1, gate_b1, gate_W2, gate_b2, W1, b1, W2, b2, W3, b3):
    out, usage, loss = pl.pallas_call(
        _moe_body,
        grid=(_NT + 1,),
        in_specs=[
            pl.BlockSpec((_N, _D), lambda i: (0, 0)),
            pl.BlockSpec((_D, _GH), lambda i: (0, 0)),
            pl.BlockSpec((1, _GH), lambda i: (0, 0)),
            pl.BlockSpec((_GH, _E), lambda i: (0, 0)),
            pl.BlockSpec((1, _E), lambda i: (0, 0)),
            pl.BlockSpec(memory_space=pl.ANY),
            pl.BlockSpec((_E, _H), lambda i: (0, 0)),
            pl.BlockSpec(memory_space=pl.ANY),
            pl.BlockSpec((_E, _H), lambda i: (0, 0)),
            pl.BlockSpec(memory_space=pl.ANY),
            pl.BlockSpec((_E, _D), lambda i: (0, 0)),
        ],
        out_specs=[
            pl.BlockSpec((_TN, _D),
                         lambda i: (jnp.maximum(i - 1, 0), 0)),
            pl.BlockSpec((1, _E), lambda i: (0, 0)),
            pl.BlockSpec((1, 1), lambda i: (0, 0)),
        ],
        out_shape=[
            jax.ShapeDtypeStruct((_N, _D), jnp.float32),
            jax.ShapeDtypeStruct((1, _E), jnp.float32),
            jax.ShapeDtypeStruct((1, 1), jnp.float32),
        ],
        scratch_shapes=[
            pltpu.VMEM((_TN, _EH), jnp.bfloat16),
            pltpu.VMEM((_N, _E), jnp.float32),
            pltpu.VMEM((_E, _D, _H), jnp.float32),
            pltpu.VMEM((_E, _H, _H), jnp.float32),
            pltpu.VMEM((_E, _H, _D), jnp.float32),
            pltpu.VMEM((_D, _EH), jnp.bfloat16),
            pltpu.VMEM((1, _EH), jnp.float32),
            pltpu.VMEM((_E // 2, _H2, _H2), jnp.bfloat16),
            pltpu.VMEM((_EH, _D), jnp.bfloat16),
            pltpu.SemaphoreType.DMA,
            pltpu.SemaphoreType.DMA,
            pltpu.SemaphoreType.DMA,
        ],
    )(x, gate_W1, gate_b1.reshape(1, _GH), gate_W2, gate_b2.reshape(1, _E),
      W1, b1, W2, b2, W3, b3)
    return out, loss.reshape(()), usage.reshape(_E)
